# baseline (speedup 1.0000x reference)
import jax
import jax.numpy as jnp
from jax.experimental import pallas as pl
from jax.experimental.pallas import tpu as pltpu

_BM = 80


def _rgcn_body(adj_ref, x_ref, w_ref, b_ref, o_ref):
    msg0 = jax.lax.dot(adj_ref[0], x_ref[...],
                       preferred_element_type=jnp.float32)
    msg1 = jax.lax.dot(adj_ref[1], x_ref[...],
                       preferred_element_type=jnp.float32)
    out = (jax.lax.dot(msg0, w_ref[0], preferred_element_type=jnp.float32)
           + jax.lax.dot(msg1, w_ref[1], preferred_element_type=jnp.float32)
           + b_ref[...])
    o_ref[...] = jnp.maximum(out, 0.0)


def kernel(node_features, adj_list, weight, bias):
    n, in_dim = node_features.shape
    r = adj_list.shape[0]
    out_dim = weight.shape[-1]
    num_m = n // _BM

    b2 = bias.reshape(1, out_dim)

    return pl.pallas_call(
        _rgcn_body,
        grid=(num_m,),
        in_specs=[
            pl.BlockSpec((r, _BM, n), lambda m: (0, m, 0)),
            pl.BlockSpec((n, in_dim), lambda m: (0, 0)),
            pl.BlockSpec((r, in_dim, out_dim), lambda m: (0, 0, 0)),
            pl.BlockSpec((1, out_dim), lambda m: (0, 0)),
        ],
        out_specs=pl.BlockSpec((_BM, out_dim), lambda m: (m, 0)),
        out_shape=jax.ShapeDtypeStruct((n, out_dim), jnp.float32),
        compiler_params=pltpu.CompilerParams(
            dimension_semantics=("arbitrary",),
        ),
    )(adj_list, node_features, weight, b2)


# speedup vs baseline: 1.1222x; 1.1222x over previous
import jax
import jax.numpy as jnp
from jax.experimental import pallas as pl
from jax.experimental.pallas import tpu as pltpu

_BM = 200
_NBUF = 4


def _rgcn_body(adj_ref, x_ref, w_ref, b_ref, o_ref, buf, acc, sems):
    n = x_ref.shape[0]
    nrel = adj_ref.shape[0]
    nslab = nrel * (n // _BM)

    def _issue(s, slot):
        r = jax.lax.rem(s, nrel)
        m = jax.lax.div(s, nrel)
        pltpu.make_async_copy(
            adj_ref.at[r, pl.ds(pl.multiple_of(m * _BM, 8), _BM), :],
            buf.at[slot],
            sems.at[slot],
        ).start()

    for s0 in range(_NBUF):
        _issue(jnp.int32(s0), jnp.int32(s0))

    def _step(s, carry):
        slot = jax.lax.rem(s, _NBUF)
        r = jax.lax.rem(s, nrel)
        m = jax.lax.div(s, nrel)
        pltpu.make_async_copy(
            adj_ref.at[r, pl.ds(pl.multiple_of(m * _BM, 8), _BM), :],
            buf.at[slot],
            sems.at[slot],
        ).wait()
        msg = jax.lax.dot(buf[slot], x_ref[...],
                          preferred_element_type=jnp.float32)
        part = jax.lax.dot(msg, w_ref[r], preferred_element_type=jnp.float32)

        @pl.when(r == 0)
        def _first():
            acc[...] = part

        @pl.when(r == nrel - 1)
        def _last():
            row = pl.multiple_of(m * _BM, 8)
            o_ref[pl.ds(row, _BM), :] = jnp.maximum(
                acc[...] + part + b_ref[...], 0.0)

        @pl.when(s + _NBUF < nslab)
        def _refill():
            _issue(s + _NBUF, slot)

        return carry

    jax.lax.fori_loop(0, nslab, _step, 0)


def kernel(node_features, adj_list, weight, bias):
    n, in_dim = node_features.shape
    r = adj_list.shape[0]
    out_dim = weight.shape[-1]

    b2 = bias.reshape(1, out_dim)

    return pl.pallas_call(
        _rgcn_body,
        in_specs=[
            pl.BlockSpec(memory_space=pl.ANY),
            pl.BlockSpec(memory_space=pltpu.VMEM),
            pl.BlockSpec(memory_space=pltpu.VMEM),
            pl.BlockSpec(memory_space=pltpu.VMEM),
        ],
        out_specs=pl.BlockSpec(memory_space=pltpu.VMEM),
        out_shape=jax.ShapeDtypeStruct((n, out_dim), jnp.float32),
        scratch_shapes=[
            pltpu.VMEM((_NBUF, _BM, n), jnp.float32),
            pltpu.VMEM((_BM, out_dim), jnp.float32),
            pltpu.SemaphoreType.DMA((_NBUF,)),
        ],
    )(adj_list, node_features, weight, b2)
